# Initial kernel scaffold; baseline (speedup 1.0000x reference)
import functools

import jax
import jax.numpy as jnp
from jax import lax
from jax.experimental import pallas as pl
from jax.experimental.pallas import tpu as pltpu
from jax.experimental.pallas import tpu_sc as plsc

N = 10000
D = 128
H = 640
E = 320000

NC = 2
NS = 16
NW = NC * NS

CH = 80
EPW = E // NW
NCHUNK = EPW // CH

ROWS_PER_TILE = 624
TAIL_START = ROWS_PER_TILE * NS
TAIL_ROWS = N - TAIL_START

_MESH = plsc.VectorSubcoreMesh(
    core_axis_name="c", subcore_axis_name="s", num_cores=NC, num_subcores=NS
)


@functools.partial(
    pl.kernel,
    out_type=jax.ShapeDtypeStruct((NC, N), jnp.float32),
    mesh=_MESH,
    scratch_types=[
        pltpu.VMEM((CH,), jnp.int32),
        pltpu.VMEM((CH,), jnp.float32),
        pltpu.VMEM_SHARED((N,), jnp.float32),
    ],
)
def _deg_kernel(dst_hbm, zeros_hbm, out_hbm, idx_v, ones_v, acc):
    c = lax.axis_index("c")
    s = lax.axis_index("s")
    wid = c * NS + s
    for i in range(CH // 16):
        ones_v[pl.ds(i * 16, 16)] = jnp.ones((16,), jnp.float32)
    pltpu.sync_copy(
        zeros_hbm.at[pl.ds(s * ROWS_PER_TILE, ROWS_PER_TILE)],
        acc.at[pl.ds(s * ROWS_PER_TILE, ROWS_PER_TILE)],
    )

    @pl.when(s == 0)
    def _():
        pltpu.sync_copy(
            zeros_hbm.at[pl.ds(TAIL_START, TAIL_ROWS)],
            acc.at[pl.ds(TAIL_START, TAIL_ROWS)],
        )

    plsc.subcore_barrier()
    base = wid * EPW

    def body(i, carry):
        pltpu.sync_copy(dst_hbm.at[pl.ds(base + i * CH, CH)], idx_v)
        pltpu.sync_copy(ones_v, acc.at[idx_v], add=True)
        return carry

    lax.fori_loop(0, NCHUNK, body, 0)
    plsc.subcore_barrier()
    pltpu.sync_copy(
        acc.at[pl.ds(s * ROWS_PER_TILE, ROWS_PER_TILE)],
        out_hbm.at[c, pl.ds(s * ROWS_PER_TILE, ROWS_PER_TILE)],
    )

    @pl.when(s == 0)
    def _():
        pltpu.sync_copy(
            acc.at[pl.ds(TAIL_START, TAIL_ROWS)],
            out_hbm.at[c, pl.ds(TAIL_START, TAIL_ROWS)],
        )


@functools.partial(
    pl.kernel,
    out_type=jax.ShapeDtypeStruct((NC, N, D), jnp.float32),
    mesh=_MESH,
    scratch_types=[
        pltpu.VMEM((CH,), jnp.int32),
        pltpu.VMEM((CH,), jnp.int32),
        pltpu.VMEM((CH, D), jnp.float32),
        pltpu.VMEM_SHARED((N, D), jnp.float32),
        pltpu.SemaphoreType.DMA,
    ],
)
def _agg_kernel(src_hbm, dst_hbm, y_hbm, zeros_hbm, out_hbm,
                src_v, dst_v, rows_v, acc, sem):
    c = lax.axis_index("c")
    s = lax.axis_index("s")
    wid = c * NS + s
    pltpu.sync_copy(
        zeros_hbm.at[pl.ds(s * ROWS_PER_TILE, ROWS_PER_TILE)],
        acc.at[pl.ds(s * ROWS_PER_TILE, ROWS_PER_TILE)],
    )

    @pl.when(s == 0)
    def _():
        pltpu.sync_copy(
            zeros_hbm.at[pl.ds(TAIL_START, TAIL_ROWS)],
            acc.at[pl.ds(TAIL_START, TAIL_ROWS)],
        )

    plsc.subcore_barrier()
    base = wid * EPW

    def body(i, carry):
        pltpu.sync_copy(src_hbm.at[pl.ds(base + i * CH, CH)], src_v)
        pltpu.sync_copy(dst_hbm.at[pl.ds(base + i * CH, CH)], dst_v)
        pltpu.async_copy(y_hbm.at[src_v], rows_v, sem).wait()
        pltpu.sync_copy(rows_v, acc.at[dst_v], add=True)
        return carry

    lax.fori_loop(0, NCHUNK, body, 0)
    plsc.subcore_barrier()
    pltpu.sync_copy(
        acc.at[pl.ds(s * ROWS_PER_TILE, ROWS_PER_TILE)],
        out_hbm.at[c, pl.ds(s * ROWS_PER_TILE, ROWS_PER_TILE)],
    )

    @pl.when(s == 0)
    def _():
        pltpu.sync_copy(
            acc.at[pl.ds(TAIL_START, TAIL_ROWS)],
            out_hbm.at[c, pl.ds(TAIL_START, TAIL_ROWS)],
        )


_GRID = 5
_R = N // _GRID


def _prep_body(hist_ref, x_ref, mask_ref, y1_ref, u_ref):
    deg = hist_ref[:, 0] + hist_ref[:, 1] + 1.0
    u = lax.rsqrt(deg)
    ub = jnp.broadcast_to(u[:, None], (_R, D))
    u_ref[...] = ub
    y1_ref[...] = x_ref[...] * mask_ref[...] * ub


def _tc_prep(hist_t, x, mask):
    return pl.pallas_call(
        _prep_body,
        grid=(_GRID,),
        in_specs=[
            pl.BlockSpec((_R, NC), lambda i: (i, 0)),
            pl.BlockSpec((_R, D), lambda i: (i, 0)),
            pl.BlockSpec((_R, D), lambda i: (i, 0)),
        ],
        out_specs=[
            pl.BlockSpec((_R, D), lambda i: (i, 0)),
            pl.BlockSpec((_R, D), lambda i: (i, 0)),
        ],
        out_shape=[
            jax.ShapeDtypeStruct((N, D), jnp.float32),
            jax.ShapeDtypeStruct((N, D), jnp.float32),
        ],
    )(hist_t, x, mask)


def _dense_body(s_ref, y1_ref, u_ref, w1_ref, b1_ref, w2_ref, y2_ref):
    t = (s_ref[0] + y1_ref[...]) * u_ref[...]
    z = jnp.dot(t, w1_ref[...], preferred_element_type=jnp.float32,
                precision=lax.Precision.HIGHEST)
    z = jnp.maximum(z + b1_ref[...][None, :], 0.0)
    h2 = jnp.dot(z, w2_ref[...], preferred_element_type=jnp.float32,
                 precision=lax.Precision.HIGHEST)
    y2_ref[...] = h2 * u_ref[...]


def _tc_dense(s_sum, y1, u, w1, b1, w2):
    return pl.pallas_call(
        _dense_body,
        grid=(_GRID,),
        in_specs=[
            pl.BlockSpec((1, _R, D), lambda i: (0, i, 0)),
            pl.BlockSpec((_R, D), lambda i: (i, 0)),
            pl.BlockSpec((_R, D), lambda i: (i, 0)),
            pl.BlockSpec((D, H), lambda i: (0, 0)),
            pl.BlockSpec((H,), lambda i: (0,)),
            pl.BlockSpec((H, D), lambda i: (0, 0)),
        ],
        out_specs=pl.BlockSpec((_R, D), lambda i: (i, 0)),
        out_shape=jax.ShapeDtypeStruct((N, D), jnp.float32),
    )(s_sum, y1, u, w1, b1, w2)


def _final_body(s_ref, y2_ref, u_ref, b2_ref, out_ref):
    t = (s_ref[0] + y2_ref[...]) * u_ref[...]
    out_ref[...] = t + b2_ref[...][None, :]


def _tc_final(s2_sum, y2, u, b2):
    return pl.pallas_call(
        _final_body,
        grid=(_GRID,),
        in_specs=[
            pl.BlockSpec((1, _R, D), lambda i: (0, i, 0)),
            pl.BlockSpec((_R, D), lambda i: (i, 0)),
            pl.BlockSpec((_R, D), lambda i: (i, 0)),
            pl.BlockSpec((D,), lambda i: (0,)),
        ],
        out_specs=pl.BlockSpec((_R, D), lambda i: (i, 0)),
        out_shape=jax.ShapeDtypeStruct((N, D), jnp.float32),
    )(s2_sum, y2, u, b2)


def _sum_body(p_ref, out_ref):
    out_ref[...] = p_ref[0] + p_ref[1]


def _tc_sum_partials(parts):
    return pl.pallas_call(
        _sum_body,
        grid=(_GRID,),
        in_specs=[pl.BlockSpec((NC, _R, D), lambda i: (0, i, 0))],
        out_specs=pl.BlockSpec((1, _R, D), lambda i: (0, i, 0)),
        out_shape=jax.ShapeDtypeStruct((1, N, D), jnp.float32),
    )(parts)


def kernel(x, edge_index, input_mask, W1, b1, W2, b2):
    src = edge_index[0]
    dst = edge_index[1]
    zeros_n = jnp.zeros((N,), jnp.float32)
    zeros_nd = jnp.zeros((N, D), jnp.float32)

    hist = _deg_kernel(dst, zeros_n)
    y1, u = _tc_prep(hist.T, x, input_mask)
    s1 = _agg_kernel(src, dst, y1, zeros_nd)
    s1_sum = _tc_sum_partials(s1)
    y2 = _tc_dense(s1_sum, y1, u, W1, b1, W2)
    s2 = _agg_kernel(src, dst, y2, zeros_nd)
    s2_sum = _tc_sum_partials(s2)
    return _tc_final(s2_sum, y2, u, b2)


# speedup vs baseline: 17.4631x; 17.4631x over previous
import functools

import jax
import jax.numpy as jnp
from jax import lax
from jax.experimental import pallas as pl
from jax.experimental.pallas import tpu as pltpu
from jax.experimental.pallas import tpu_sc as plsc

N = 10000
D = 128
H = 640
E = 320000

NC = 2
NS = 16
NW = NC * NS

CH = 80
EPW = E // NW
NCHUNK = EPW // CH

ROWS_PER_TILE = 624
TAIL_START = ROWS_PER_TILE * NS
TAIL_ROWS = N - TAIL_START
STAGE_ROWS = 104

_MESH = plsc.VectorSubcoreMesh(
    core_axis_name="c", subcore_axis_name="s", num_cores=NC, num_subcores=NS
)


@functools.partial(
    pl.kernel,
    out_type=jax.ShapeDtypeStruct((NC * N,), jnp.float32),
    mesh=_MESH,
    scratch_types=[
        pltpu.VMEM((CH,), jnp.int32),
        pltpu.VMEM((CH,), jnp.float32),
        pltpu.VMEM((ROWS_PER_TILE,), jnp.float32),
        pltpu.VMEM_SHARED((N,), jnp.float32),
    ],
)
def _deg_kernel(dst_hbm, zeros_hbm, out_hbm, idx_v, ones_v, stage, acc):
    c = lax.axis_index("c")
    s = lax.axis_index("s")
    wid = c * NS + s
    for i in range(CH // 16):
        ones_v[pl.ds(i * 16, 16)] = jnp.ones((16,), jnp.float32)
    pltpu.sync_copy(zeros_hbm.at[pl.ds(0, ROWS_PER_TILE)], stage)
    pltpu.sync_copy(stage, acc.at[pl.ds(s * ROWS_PER_TILE, ROWS_PER_TILE)])

    @pl.when(s == 0)
    def _():
        pltpu.sync_copy(stage.at[pl.ds(0, TAIL_ROWS)],
                        acc.at[pl.ds(TAIL_START, TAIL_ROWS)])

    plsc.subcore_barrier()
    base = wid * EPW

    def body(i, carry):
        pltpu.sync_copy(dst_hbm.at[pl.ds(base + i * CH, CH)], idx_v)
        pltpu.sync_copy(ones_v, acc.at[idx_v], add=True)
        return carry

    lax.fori_loop(0, NCHUNK, body, 0)
    plsc.subcore_barrier()
    pltpu.sync_copy(acc.at[pl.ds(s * ROWS_PER_TILE, ROWS_PER_TILE)], stage)
    pltpu.sync_copy(stage,
                    out_hbm.at[pl.ds(c * N + s * ROWS_PER_TILE,
                                     ROWS_PER_TILE)])

    @pl.when(s == 0)
    def _():
        pltpu.sync_copy(acc.at[pl.ds(TAIL_START, TAIL_ROWS)],
                        stage.at[pl.ds(0, TAIL_ROWS)])
        pltpu.sync_copy(stage.at[pl.ds(0, TAIL_ROWS)],
                        out_hbm.at[pl.ds(c * N + TAIL_START, TAIL_ROWS)])


@functools.partial(
    pl.kernel,
    out_type=jax.ShapeDtypeStruct((NC, N, D), jnp.float32),
    mesh=_MESH,
    scratch_types=[
        pltpu.VMEM((CH,), jnp.int32),
        pltpu.VMEM((CH,), jnp.int32),
        pltpu.VMEM((CH, D), jnp.float32),
        pltpu.VMEM((STAGE_ROWS, D), jnp.float32),
        pltpu.VMEM_SHARED((N, D), jnp.float32),
        pltpu.SemaphoreType.DMA,
    ],
)
def _agg_kernel(src_hbm, dst_hbm, y_hbm, zeros_hbm, out_hbm,
                src_v, dst_v, rows_v, stage, acc, sem):
    c = lax.axis_index("c")
    s = lax.axis_index("s")
    wid = c * NS + s
    pltpu.sync_copy(zeros_hbm.at[pl.ds(0, STAGE_ROWS)], stage)
    for k in range(ROWS_PER_TILE // STAGE_ROWS):
        pltpu.sync_copy(
            stage,
            acc.at[pl.ds(s * ROWS_PER_TILE + k * STAGE_ROWS, STAGE_ROWS)],
        )

    @pl.when(s == 0)
    def _():
        pltpu.sync_copy(stage.at[pl.ds(0, TAIL_ROWS)],
                        acc.at[pl.ds(TAIL_START, TAIL_ROWS)])

    plsc.subcore_barrier()
    base = wid * EPW

    def body(i, carry):
        pltpu.sync_copy(src_hbm.at[pl.ds(base + i * CH, CH)], src_v)
        pltpu.sync_copy(dst_hbm.at[pl.ds(base + i * CH, CH)], dst_v)
        pltpu.async_copy(y_hbm.at[src_v], rows_v, sem).wait()
        pltpu.sync_copy(rows_v, acc.at[dst_v], add=True)
        return carry

    lax.fori_loop(0, NCHUNK, body, 0)
    plsc.subcore_barrier()
    for k in range(ROWS_PER_TILE // STAGE_ROWS):
        r0 = k * STAGE_ROWS
        pltpu.sync_copy(
            acc.at[pl.ds(s * ROWS_PER_TILE + r0, STAGE_ROWS)], stage)
        pltpu.sync_copy(
            stage, out_hbm.at[c, pl.ds(s * ROWS_PER_TILE + r0, STAGE_ROWS)])

    @pl.when(s == 0)
    def _():
        pltpu.sync_copy(acc.at[pl.ds(TAIL_START, TAIL_ROWS)],
                        stage.at[pl.ds(0, TAIL_ROWS)])
        pltpu.sync_copy(stage.at[pl.ds(0, TAIL_ROWS)],
                        out_hbm.at[c, pl.ds(TAIL_START, TAIL_ROWS)])


_GRID = 5
_R = N // _GRID


def _prep_body(hist_ref, x_ref, mask_ref, y1_ref, u_ref):
    deg = hist_ref[:, 0] + hist_ref[:, 1] + 1.0
    u = lax.rsqrt(deg)
    ub = jnp.broadcast_to(u[:, None], (_R, D))
    u_ref[...] = ub
    y1_ref[...] = x_ref[...] * mask_ref[...] * ub


def _tc_prep(hist_t, x, mask):
    return pl.pallas_call(
        _prep_body,
        grid=(_GRID,),
        in_specs=[
            pl.BlockSpec((_R, NC), lambda i: (i, 0)),
            pl.BlockSpec((_R, D), lambda i: (i, 0)),
            pl.BlockSpec((_R, D), lambda i: (i, 0)),
        ],
        out_specs=[
            pl.BlockSpec((_R, D), lambda i: (i, 0)),
            pl.BlockSpec((_R, D), lambda i: (i, 0)),
        ],
        out_shape=[
            jax.ShapeDtypeStruct((N, D), jnp.float32),
            jax.ShapeDtypeStruct((N, D), jnp.float32),
        ],
    )(hist_t, x, mask)


def _dense_body(s_ref, y1_ref, u_ref, w1_ref, b1_ref, w2_ref, y2_ref):
    t = (s_ref[0] + s_ref[1] + y1_ref[...]) * u_ref[...]
    z = jnp.dot(t, w1_ref[...], preferred_element_type=jnp.float32,
                precision=lax.Precision.HIGHEST)
    z = jnp.maximum(z + b1_ref[...][None, :], 0.0)
    h2 = jnp.dot(z, w2_ref[...], preferred_element_type=jnp.float32,
                 precision=lax.Precision.HIGHEST)
    y2_ref[...] = h2 * u_ref[...]


def _tc_dense(s_sum, y1, u, w1, b1, w2):
    return pl.pallas_call(
        _dense_body,
        grid=(_GRID,),
        in_specs=[
            pl.BlockSpec((NC, _R, D), lambda i: (0, i, 0)),
            pl.BlockSpec((_R, D), lambda i: (i, 0)),
            pl.BlockSpec((_R, D), lambda i: (i, 0)),
            pl.BlockSpec((D, H), lambda i: (0, 0)),
            pl.BlockSpec((H,), lambda i: (0,)),
            pl.BlockSpec((H, D), lambda i: (0, 0)),
        ],
        out_specs=pl.BlockSpec((_R, D), lambda i: (i, 0)),
        out_shape=jax.ShapeDtypeStruct((N, D), jnp.float32),
    )(s_sum, y1, u, w1, b1, w2)


def _final_body(s_ref, y2_ref, u_ref, b2_ref, out_ref):
    t = (s_ref[0] + s_ref[1] + y2_ref[...]) * u_ref[...]
    out_ref[...] = t + b2_ref[...][None, :]


def _tc_final(s2_sum, y2, u, b2):
    return pl.pallas_call(
        _final_body,
        grid=(_GRID,),
        in_specs=[
            pl.BlockSpec((NC, _R, D), lambda i: (0, i, 0)),
            pl.BlockSpec((_R, D), lambda i: (i, 0)),
            pl.BlockSpec((_R, D), lambda i: (i, 0)),
            pl.BlockSpec((D,), lambda i: (0,)),
        ],
        out_specs=pl.BlockSpec((_R, D), lambda i: (i, 0)),
        out_shape=jax.ShapeDtypeStruct((N, D), jnp.float32),
    )(s2_sum, y2, u, b2)


def kernel(x, edge_index, input_mask, W1, b1, W2, b2):
    src = edge_index[0]
    dst = edge_index[1]
    zeros_n = jnp.zeros((N,), jnp.float32)
    zeros_nd = jnp.zeros((N, D), jnp.float32)

    hist = _deg_kernel(dst, zeros_n)
    y1, u = _tc_prep(hist.reshape(NC, N).T, x, input_mask)
    s1 = _agg_kernel(src, dst, y1, zeros_nd)
    y2 = _tc_dense(s1, y1, u, W1, b1, W2)
    s2 = _agg_kernel(src, dst, y2, zeros_nd)
    return _tc_final(s2, y2, u, b2)


# speedup vs baseline: 36.3730x; 2.0829x over previous
import functools

import jax
import jax.numpy as jnp
from jax import lax
from jax.experimental import pallas as pl
from jax.experimental.pallas import tpu as pltpu
from jax.experimental.pallas import tpu_sc as plsc

N = 10000
D = 128
H = 640
E = 320000

NC = 2
NS = 16
NW = NC * NS

CH = 80
EPW = E // NW
NCHUNK = EPW // CH

ROWS_PER_TILE = 624
TAIL_START = ROWS_PER_TILE * NS
TAIL_ROWS = N - TAIL_START
STAGE_ROWS = 104

_MESH = plsc.VectorSubcoreMesh(
    core_axis_name="c", subcore_axis_name="s", num_cores=NC, num_subcores=NS
)


@functools.partial(
    pl.kernel,
    out_type=jax.ShapeDtypeStruct((NC * N,), jnp.float32),
    mesh=_MESH,
    scratch_types=[
        pltpu.VMEM((CH,), jnp.int32),
        pltpu.VMEM((CH,), jnp.int32),
        pltpu.VMEM((CH,), jnp.float32),
        pltpu.VMEM((ROWS_PER_TILE,), jnp.float32),
        pltpu.VMEM_SHARED((N,), jnp.float32),
        pltpu.SemaphoreType.DMA,
        pltpu.SemaphoreType.DMA,
    ],
)
def _deg_kernel(dst_hbm, zeros_hbm, out_hbm, id0, id1, ones_v, stage, acc,
                sem0, sem1):
    c = lax.axis_index("c")
    s = lax.axis_index("s")
    wid = c * NS + s
    base = wid * EPW
    for i in range(CH // 16):
        ones_v[pl.ds(i * 16, 16)] = jnp.ones((16,), jnp.float32)
    pltpu.sync_copy(zeros_hbm.at[pl.ds(0, ROWS_PER_TILE)], stage)
    pltpu.sync_copy(stage, acc.at[pl.ds(s * ROWS_PER_TILE, ROWS_PER_TILE)])

    @pl.when(s == 0)
    def _():
        pltpu.sync_copy(stage.at[pl.ds(0, TAIL_ROWS)],
                        acc.at[pl.ds(TAIL_START, TAIL_ROWS)])

    plsc.subcore_barrier()
    pltpu.async_copy(dst_hbm.at[pl.ds(base, CH)], id0, sem0)

    def body(k, carry):
        g = 2 * k
        pltpu.async_copy(dst_hbm.at[pl.ds(base + (g + 1) * CH, CH)], id1,
                         sem1)
        pltpu.make_async_copy(dst_hbm.at[pl.ds(base, CH)], id0, sem0).wait()
        pltpu.sync_copy(ones_v, acc.at[id0], add=True)
        pltpu.async_copy(dst_hbm.at[pl.ds(base + (g + 2) * CH, CH)], id0,
                         sem0)
        pltpu.make_async_copy(dst_hbm.at[pl.ds(base, CH)], id1, sem1).wait()
        pltpu.sync_copy(ones_v, acc.at[id1], add=True)
        return carry

    lax.fori_loop(0, (NCHUNK - 1) // 2, body, 0)
    pltpu.make_async_copy(dst_hbm.at[pl.ds(base, CH)], id0, sem0).wait()
    pltpu.sync_copy(ones_v, acc.at[id0], add=True)
    plsc.subcore_barrier()
    pltpu.sync_copy(acc.at[pl.ds(s * ROWS_PER_TILE, ROWS_PER_TILE)], stage)
    pltpu.sync_copy(stage,
                    out_hbm.at[pl.ds(c * N + s * ROWS_PER_TILE,
                                     ROWS_PER_TILE)])

    @pl.when(s == 0)
    def _():
        pltpu.sync_copy(acc.at[pl.ds(TAIL_START, TAIL_ROWS)],
                        stage.at[pl.ds(0, TAIL_ROWS)])
        pltpu.sync_copy(stage.at[pl.ds(0, TAIL_ROWS)],
                        out_hbm.at[pl.ds(c * N + TAIL_START, TAIL_ROWS)])


@functools.partial(
    pl.kernel,
    out_type=jax.ShapeDtypeStruct((NC, N, D), jnp.float32),
    mesh=_MESH,
    scratch_types=[
        pltpu.VMEM((EPW,), jnp.int32),
        pltpu.VMEM((CH,), jnp.int32),
        pltpu.VMEM((CH,), jnp.int32),
        pltpu.VMEM((CH, D), jnp.float32),
        pltpu.VMEM((CH, D), jnp.float32),
        pltpu.VMEM((STAGE_ROWS, D), jnp.float32),
        pltpu.VMEM_SHARED((N, D), jnp.float32),
        pltpu.SemaphoreType.DMA,
        pltpu.SemaphoreType.DMA,
        pltpu.SemaphoreType.DMA,
        pltpu.SemaphoreType.DMA,
    ],
)
def _agg_kernel(src_hbm, dst_hbm, y_hbm, zeros_hbm, out_hbm,
                src_v, id0, id1, buf0, buf1, stage, acc,
                semg0, semg1, semi0, semi1):
    c = lax.axis_index("c")
    s = lax.axis_index("s")
    wid = c * NS + s
    base = wid * EPW
    pltpu.sync_copy(src_hbm.at[pl.ds(base, EPW)], src_v)
    pltpu.sync_copy(zeros_hbm.at[pl.ds(0, STAGE_ROWS)], stage)
    for k in range(ROWS_PER_TILE // STAGE_ROWS):
        pltpu.sync_copy(
            stage,
            acc.at[pl.ds(s * ROWS_PER_TILE + k * STAGE_ROWS, STAGE_ROWS)],
        )

    @pl.when(s == 0)
    def _():
        pltpu.sync_copy(stage.at[pl.ds(0, TAIL_ROWS)],
                        acc.at[pl.ds(TAIL_START, TAIL_ROWS)])

    plsc.subcore_barrier()

    pltpu.async_copy(y_hbm.at[src_v.at[pl.ds(0, CH)]], buf0, semg0)
    pltpu.async_copy(dst_hbm.at[pl.ds(base, CH)], id0, semi0)

    def body(k, carry):
        g = 2 * k
        pltpu.async_copy(y_hbm.at[src_v.at[pl.ds((g + 1) * CH, CH)]], buf1,
                         semg1)
        pltpu.async_copy(dst_hbm.at[pl.ds(base + (g + 1) * CH, CH)], id1,
                         semi1)
        pltpu.make_async_copy(y_hbm.at[src_v.at[pl.ds(0, CH)]], buf0,
                              semg0).wait()
        pltpu.make_async_copy(dst_hbm.at[pl.ds(base, CH)], id0,
                              semi0).wait()
        pltpu.sync_copy(buf0, acc.at[id0], add=True)
        pltpu.async_copy(y_hbm.at[src_v.at[pl.ds((g + 2) * CH, CH)]], buf0,
                         semg0)
        pltpu.async_copy(dst_hbm.at[pl.ds(base + (g + 2) * CH, CH)], id0,
                         semi0)
        pltpu.make_async_copy(y_hbm.at[src_v.at[pl.ds(0, CH)]], buf1,
                              semg1).wait()
        pltpu.make_async_copy(dst_hbm.at[pl.ds(base, CH)], id1,
                              semi1).wait()
        pltpu.sync_copy(buf1, acc.at[id1], add=True)
        return carry

    lax.fori_loop(0, (NCHUNK - 1) // 2, body, 0)
    pltpu.make_async_copy(y_hbm.at[src_v.at[pl.ds(0, CH)]], buf0,
                          semg0).wait()
    pltpu.make_async_copy(dst_hbm.at[pl.ds(base, CH)], id0, semi0).wait()
    pltpu.sync_copy(buf0, acc.at[id0], add=True)
    plsc.subcore_barrier()
    for k in range(ROWS_PER_TILE // STAGE_ROWS):
        r0 = k * STAGE_ROWS
        pltpu.sync_copy(
            acc.at[pl.ds(s * ROWS_PER_TILE + r0, STAGE_ROWS)], stage)
        pltpu.sync_copy(
            stage, out_hbm.at[c, pl.ds(s * ROWS_PER_TILE + r0, STAGE_ROWS)])

    @pl.when(s == 0)
    def _():
        pltpu.sync_copy(acc.at[pl.ds(TAIL_START, TAIL_ROWS)],
                        stage.at[pl.ds(0, TAIL_ROWS)])
        pltpu.sync_copy(stage.at[pl.ds(0, TAIL_ROWS)],
                        out_hbm.at[c, pl.ds(TAIL_START, TAIL_ROWS)])


_GRID = 5
_R = N // _GRID


def _prep_body(hist_ref, x_ref, mask_ref, y1_ref, u_ref):
    deg = hist_ref[:, 0] + hist_ref[:, 1] + 1.0
    u = lax.rsqrt(deg)
    ub = jnp.broadcast_to(u[:, None], (_R, D))
    u_ref[...] = ub
    y1_ref[...] = x_ref[...] * mask_ref[...] * ub


def _tc_prep(hist_t, x, mask):
    return pl.pallas_call(
        _prep_body,
        grid=(_GRID,),
        in_specs=[
            pl.BlockSpec((_R, NC), lambda i: (i, 0)),
            pl.BlockSpec((_R, D), lambda i: (i, 0)),
            pl.BlockSpec((_R, D), lambda i: (i, 0)),
        ],
        out_specs=[
            pl.BlockSpec((_R, D), lambda i: (i, 0)),
            pl.BlockSpec((_R, D), lambda i: (i, 0)),
        ],
        out_shape=[
            jax.ShapeDtypeStruct((N, D), jnp.float32),
            jax.ShapeDtypeStruct((N, D), jnp.float32),
        ],
    )(hist_t, x, mask)


def _dense_body(s_ref, y1_ref, u_ref, w1_ref, b1_ref, w2_ref, y2_ref):
    t = (s_ref[0] + s_ref[1] + y1_ref[...]) * u_ref[...]
    z = jnp.dot(t, w1_ref[...], preferred_element_type=jnp.float32,
                precision=lax.Precision.HIGHEST)
    z = jnp.maximum(z + b1_ref[...][None, :], 0.0)
    h2 = jnp.dot(z, w2_ref[...], preferred_element_type=jnp.float32,
                 precision=lax.Precision.HIGHEST)
    y2_ref[...] = h2 * u_ref[...]


def _tc_dense(s_sum, y1, u, w1, b1, w2):
    return pl.pallas_call(
        _dense_body,
        grid=(_GRID,),
        in_specs=[
            pl.BlockSpec((NC, _R, D), lambda i: (0, i, 0)),
            pl.BlockSpec((_R, D), lambda i: (i, 0)),
            pl.BlockSpec((_R, D), lambda i: (i, 0)),
            pl.BlockSpec((D, H), lambda i: (0, 0)),
            pl.BlockSpec((H,), lambda i: (0,)),
            pl.BlockSpec((H, D), lambda i: (0, 0)),
        ],
        out_specs=pl.BlockSpec((_R, D), lambda i: (i, 0)),
        out_shape=jax.ShapeDtypeStruct((N, D), jnp.float32),
    )(s_sum, y1, u, w1, b1, w2)


def _final_body(s_ref, y2_ref, u_ref, b2_ref, out_ref):
    t = (s_ref[0] + s_ref[1] + y2_ref[...]) * u_ref[...]
    out_ref[...] = t + b2_ref[...][None, :]


def _tc_final(s2_sum, y2, u, b2):
    return pl.pallas_call(
        _final_body,
        grid=(_GRID,),
        in_specs=[
            pl.BlockSpec((NC, _R, D), lambda i: (0, i, 0)),
            pl.BlockSpec((_R, D), lambda i: (i, 0)),
            pl.BlockSpec((_R, D), lambda i: (i, 0)),
            pl.BlockSpec((D,), lambda i: (0,)),
        ],
        out_specs=pl.BlockSpec((_R, D), lambda i: (i, 0)),
        out_shape=jax.ShapeDtypeStruct((N, D), jnp.float32),
    )(s2_sum, y2, u, b2)


def kernel(x, edge_index, input_mask, W1, b1, W2, b2):
    src = edge_index[0]
    dst = edge_index[1]
    zeros_n = jnp.zeros((N,), jnp.float32)
    zeros_nd = jnp.zeros((N, D), jnp.float32)

    hist = _deg_kernel(dst, zeros_n)
    y1, u = _tc_prep(hist.reshape(NC, N).T, x, input_mask)
    s1 = _agg_kernel(src, dst, y1, zeros_nd)
    y2 = _tc_dense(s1, y1, u, W1, b1, W2)
    s2 = _agg_kernel(src, dst, y2, zeros_nd)
    return _tc_final(s2, y2, u, b2)


# speedup vs baseline: 43.0422x; 1.1834x over previous
import functools

import jax
import jax.numpy as jnp
from jax import lax
from jax.experimental import pallas as pl
from jax.experimental.pallas import tpu as pltpu
from jax.experimental.pallas import tpu_sc as plsc

N = 10000
D = 128
H = 640
E = 320000

NC = 2
NS = 16
NW = NC * NS

CH = 80
EPW = E // NW
NCHUNK = EPW // CH

ROWS_PER_TILE = 624
TAIL_START = ROWS_PER_TILE * NS
TAIL_ROWS = N - TAIL_START
NBUF = 4
ABUF = 3

_MESH = plsc.VectorSubcoreMesh(
    core_axis_name="c", subcore_axis_name="s", num_cores=NC, num_subcores=NS
)


@functools.partial(
    pl.kernel,
    out_type=jax.ShapeDtypeStruct((NC * N,), jnp.float32),
    mesh=_MESH,
    scratch_types=[
        [pltpu.VMEM((CH,), jnp.int32)] * NBUF,
        pltpu.VMEM((CH,), jnp.float32),
        pltpu.VMEM((ROWS_PER_TILE,), jnp.float32),
        pltpu.VMEM_SHARED((N,), jnp.float32),
        [pltpu.SemaphoreType.DMA] * NBUF,
    ],
)
def _deg_kernel(dst_hbm, zeros_hbm, out_hbm, ids, ones_v, stage, acc, sems):
    c = lax.axis_index("c")
    s = lax.axis_index("s")
    wid = c * NS + s
    base = wid * EPW
    for i in range(CH // 16):
        ones_v[pl.ds(i * 16, 16)] = jnp.ones((16,), jnp.float32)
    pltpu.sync_copy(zeros_hbm.at[pl.ds(0, ROWS_PER_TILE)], stage)
    pltpu.sync_copy(stage, acc.at[pl.ds(s * ROWS_PER_TILE, ROWS_PER_TILE)])

    @pl.when(s == 0)
    def _():
        pltpu.sync_copy(stage.at[pl.ds(0, TAIL_ROWS)],
                        acc.at[pl.ds(TAIL_START, TAIL_ROWS)])

    plsc.subcore_barrier()
    for b in range(NBUF):
        pltpu.async_copy(dst_hbm.at[pl.ds(base + b * CH, CH)], ids[b],
                         sems[b])

    def body(k, carry):
        for b in range(NBUF):
            g = NBUF * k + b
            pltpu.make_async_copy(dst_hbm.at[pl.ds(base, CH)], ids[b],
                                  sems[b]).wait()
            pltpu.sync_copy(ones_v, acc.at[ids[b]], add=True)

            @pl.when(g + NBUF < NCHUNK)
            def _():
                pltpu.async_copy(
                    dst_hbm.at[pl.ds(base + (g + NBUF) * CH, CH)], ids[b],
                    sems[b])

        return carry

    lax.fori_loop(0, NCHUNK // NBUF, body, 0)
    for b in range(NCHUNK % NBUF):
        g = (NCHUNK // NBUF) * NBUF + b
        pltpu.make_async_copy(dst_hbm.at[pl.ds(base, CH)], ids[b],
                              sems[b]).wait()
        pltpu.sync_copy(ones_v, acc.at[ids[b]], add=True)
    plsc.subcore_barrier()
    pltpu.sync_copy(acc.at[pl.ds(s * ROWS_PER_TILE, ROWS_PER_TILE)], stage)
    pltpu.sync_copy(stage,
                    out_hbm.at[pl.ds(c * N + s * ROWS_PER_TILE,
                                     ROWS_PER_TILE)])

    @pl.when(s == 0)
    def _():
        pltpu.sync_copy(acc.at[pl.ds(TAIL_START, TAIL_ROWS)],
                        stage.at[pl.ds(0, TAIL_ROWS)])
        pltpu.sync_copy(stage.at[pl.ds(0, TAIL_ROWS)],
                        out_hbm.at[pl.ds(c * N + TAIL_START, TAIL_ROWS)])


@functools.partial(
    pl.kernel,
    out_type=jax.ShapeDtypeStruct((NC, N, D), jnp.float32),
    mesh=_MESH,
    scratch_types=[
        pltpu.VMEM((EPW,), jnp.int32),
        [pltpu.VMEM((CH,), jnp.int32)] * ABUF,
        [pltpu.VMEM((CH, D), jnp.float32)] * ABUF,
        pltpu.VMEM_SHARED((N, D), jnp.float32),
        [pltpu.SemaphoreType.DMA] * ABUF,
        [pltpu.SemaphoreType.DMA] * ABUF,
    ],
)
def _agg_kernel(src_hbm, dst_hbm, y_hbm, zeros_hbm, out_hbm,
                src_v, ids, bufs, acc, semg, semi):
    c = lax.axis_index("c")
    s = lax.axis_index("s")
    wid = c * NS + s
    base = wid * EPW
    pltpu.sync_copy(src_hbm.at[pl.ds(base, EPW)], src_v)
    pltpu.sync_copy(zeros_hbm.at[pl.ds(0, CH)], bufs[0])
    for k in range(ROWS_PER_TILE // CH):
        pltpu.sync_copy(bufs[0],
                        acc.at[pl.ds(s * ROWS_PER_TILE + k * CH, CH)])
    pltpu.sync_copy(
        bufs[0].at[pl.ds(0, ROWS_PER_TILE % CH)],
        acc.at[pl.ds(s * ROWS_PER_TILE + (ROWS_PER_TILE // CH) * CH,
                     ROWS_PER_TILE % CH)])

    @pl.when(s == 0)
    def _():
        pltpu.sync_copy(bufs[0].at[pl.ds(0, TAIL_ROWS)],
                        acc.at[pl.ds(TAIL_START, TAIL_ROWS)])

    plsc.subcore_barrier()

    for b in range(ABUF):
        pltpu.async_copy(y_hbm.at[src_v.at[pl.ds(b * CH, CH)]], bufs[b],
                         semg[b])
        pltpu.async_copy(dst_hbm.at[pl.ds(base + b * CH, CH)], ids[b],
                         semi[b])

    def body(k, carry):
        for b in range(ABUF):
            g = ABUF * k + b
            pltpu.make_async_copy(y_hbm.at[src_v.at[pl.ds(0, CH)]], bufs[b],
                                  semg[b]).wait()
            pltpu.make_async_copy(dst_hbm.at[pl.ds(base, CH)], ids[b],
                                  semi[b]).wait()
            pltpu.sync_copy(bufs[b], acc.at[ids[b]], add=True)

            @pl.when(g + ABUF < NCHUNK)
            def _():
                pltpu.async_copy(
                    y_hbm.at[src_v.at[pl.ds((g + ABUF) * CH, CH)]], bufs[b],
                    semg[b])
                pltpu.async_copy(
                    dst_hbm.at[pl.ds(base + (g + ABUF) * CH, CH)], ids[b],
                    semi[b])

        return carry

    lax.fori_loop(0, NCHUNK // ABUF, body, 0)
    for b in range(NCHUNK % ABUF):
        pltpu.make_async_copy(y_hbm.at[src_v.at[pl.ds(0, CH)]], bufs[b],
                              semg[b]).wait()
        pltpu.make_async_copy(dst_hbm.at[pl.ds(base, CH)], ids[b],
                              semi[b]).wait()
        pltpu.sync_copy(bufs[b], acc.at[ids[b]], add=True)
    plsc.subcore_barrier()
    for k in range(ROWS_PER_TILE // CH):
        b = k % ABUF
        r0 = s * ROWS_PER_TILE + k * CH
        pltpu.sync_copy(acc.at[pl.ds(r0, CH)], bufs[b])
        pltpu.sync_copy(bufs[b], out_hbm.at[c, pl.ds(r0, CH)])
    rrem = ROWS_PER_TILE % CH
    rbase = s * ROWS_PER_TILE + (ROWS_PER_TILE // CH) * CH
    pltpu.sync_copy(acc.at[pl.ds(rbase, rrem)], bufs[0].at[pl.ds(0, rrem)])
    pltpu.sync_copy(bufs[0].at[pl.ds(0, rrem)],
                    out_hbm.at[c, pl.ds(rbase, rrem)])

    @pl.when(s == 0)
    def _():
        pltpu.sync_copy(acc.at[pl.ds(TAIL_START, TAIL_ROWS)],
                        bufs[1].at[pl.ds(0, TAIL_ROWS)])
        pltpu.sync_copy(bufs[1].at[pl.ds(0, TAIL_ROWS)],
                        out_hbm.at[c, pl.ds(TAIL_START, TAIL_ROWS)])


_GRID = 5
_R = N // _GRID


def _prep_body(hist_ref, x_ref, mask_ref, y1_ref, u_ref):
    deg = hist_ref[:, 0] + hist_ref[:, 1] + 1.0
    u = lax.rsqrt(deg)
    ub = jnp.broadcast_to(u[:, None], (_R, D))
    u_ref[...] = ub
    y1_ref[...] = x_ref[...] * mask_ref[...] * ub


def _tc_prep(hist_t, x, mask):
    return pl.pallas_call(
        _prep_body,
        grid=(_GRID,),
        in_specs=[
            pl.BlockSpec((_R, NC), lambda i: (i, 0)),
            pl.BlockSpec((_R, D), lambda i: (i, 0)),
            pl.BlockSpec((_R, D), lambda i: (i, 0)),
        ],
        out_specs=[
            pl.BlockSpec((_R, D), lambda i: (i, 0)),
            pl.BlockSpec((_R, D), lambda i: (i, 0)),
        ],
        out_shape=[
            jax.ShapeDtypeStruct((N, D), jnp.float32),
            jax.ShapeDtypeStruct((N, D), jnp.float32),
        ],
    )(hist_t, x, mask)


def _dense_body(s_ref, y1_ref, u_ref, w1_ref, b1_ref, w2_ref, y2_ref):
    t = (s_ref[0] + s_ref[1] + y1_ref[...]) * u_ref[...]
    z = jnp.dot(t, w1_ref[...], preferred_element_type=jnp.float32,
                precision=lax.Precision.HIGHEST)
    z = jnp.maximum(z + b1_ref[...][None, :], 0.0)
    h2 = jnp.dot(z, w2_ref[...], preferred_element_type=jnp.float32,
                 precision=lax.Precision.HIGHEST)
    y2_ref[...] = h2 * u_ref[...]


def _tc_dense(s_sum, y1, u, w1, b1, w2):
    return pl.pallas_call(
        _dense_body,
        grid=(_GRID,),
        in_specs=[
            pl.BlockSpec((NC, _R, D), lambda i: (0, i, 0)),
            pl.BlockSpec((_R, D), lambda i: (i, 0)),
            pl.BlockSpec((_R, D), lambda i: (i, 0)),
            pl.BlockSpec((D, H), lambda i: (0, 0)),
            pl.BlockSpec((H,), lambda i: (0,)),
            pl.BlockSpec((H, D), lambda i: (0, 0)),
        ],
        out_specs=pl.BlockSpec((_R, D), lambda i: (i, 0)),
        out_shape=jax.ShapeDtypeStruct((N, D), jnp.float32),
    )(s_sum, y1, u, w1, b1, w2)


def _final_body(s_ref, y2_ref, u_ref, b2_ref, out_ref):
    t = (s_ref[0] + s_ref[1] + y2_ref[...]) * u_ref[...]
    out_ref[...] = t + b2_ref[...][None, :]


def _tc_final(s2_sum, y2, u, b2):
    return pl.pallas_call(
        _final_body,
        grid=(_GRID,),
        in_specs=[
            pl.BlockSpec((NC, _R, D), lambda i: (0, i, 0)),
            pl.BlockSpec((_R, D), lambda i: (i, 0)),
            pl.BlockSpec((_R, D), lambda i: (i, 0)),
            pl.BlockSpec((D,), lambda i: (0,)),
        ],
        out_specs=pl.BlockSpec((_R, D), lambda i: (i, 0)),
        out_shape=jax.ShapeDtypeStruct((N, D), jnp.float32),
    )(s2_sum, y2, u, b2)


def kernel(x, edge_index, input_mask, W1, b1, W2, b2):
    src = edge_index[0]
    dst = edge_index[1]
    zeros_n = jnp.zeros((N,), jnp.float32)
    zeros_nd = jnp.zeros((N, D), jnp.float32)

    hist = _deg_kernel(dst, zeros_n)
    y1, u = _tc_prep(hist.reshape(NC, N).T, x, input_mask)
    s1 = _agg_kernel(src, dst, y1, zeros_nd)
    y2 = _tc_dense(s1, y1, u, W1, b1, W2)
    s2 = _agg_kernel(src, dst, y2, zeros_nd)
    return _tc_final(s2, y2, u, b2)


# speedup vs baseline: 48.7912x; 1.1336x over previous
import functools

import jax
import jax.numpy as jnp
from jax import lax
from jax.experimental import pallas as pl
from jax.experimental.pallas import tpu as pltpu
from jax.experimental.pallas import tpu_sc as plsc

N = 10000
D = 128
H = 640
E = 320000

NC = 2
NS = 16
NW = NC * NS

CH = 80
EPW = E // NW
NCHUNK = EPW // CH

ROWS_PER_TILE = 624
TAIL_START = ROWS_PER_TILE * NS
TAIL_ROWS = N - TAIL_START
NBUF = 4
ABUF = 3

_MESH = plsc.VectorSubcoreMesh(
    core_axis_name="c", subcore_axis_name="s", num_cores=NC, num_subcores=NS
)


@functools.partial(
    pl.kernel,
    out_type=jax.ShapeDtypeStruct((NC * N,), jnp.float32),
    mesh=_MESH,
    scratch_types=[
        [pltpu.VMEM((CH,), jnp.int32)] * NBUF,
        pltpu.VMEM((CH,), jnp.float32),
        pltpu.VMEM((ROWS_PER_TILE,), jnp.float32),
        pltpu.VMEM_SHARED((N,), jnp.float32),
        [pltpu.SemaphoreType.DMA] * NBUF,
    ],
)
def _deg_kernel(dst_hbm, zeros_hbm, out_hbm, ids, ones_v, stage, acc, sems):
    c = lax.axis_index("c")
    s = lax.axis_index("s")
    wid = c * NS + s
    base = wid * EPW
    for i in range(CH // 16):
        ones_v[pl.ds(i * 16, 16)] = jnp.ones((16,), jnp.float32)
    pltpu.sync_copy(zeros_hbm.at[pl.ds(0, ROWS_PER_TILE)], stage)
    pltpu.sync_copy(stage, acc.at[pl.ds(s * ROWS_PER_TILE, ROWS_PER_TILE)])

    @pl.when(s == 0)
    def _():
        pltpu.sync_copy(stage.at[pl.ds(0, TAIL_ROWS)],
                        acc.at[pl.ds(TAIL_START, TAIL_ROWS)])

    plsc.subcore_barrier()
    for b in range(NBUF):
        pltpu.async_copy(dst_hbm.at[pl.ds(base + b * CH, CH)], ids[b],
                         sems[b])

    def body(k, carry):
        for b in range(NBUF):
            g = NBUF * k + b
            pltpu.make_async_copy(dst_hbm.at[pl.ds(base, CH)], ids[b],
                                  sems[b]).wait()
            pltpu.sync_copy(ones_v, acc.at[ids[b]], add=True)

            @pl.when(g + NBUF < NCHUNK)
            def _():
                pltpu.async_copy(
                    dst_hbm.at[pl.ds(base + (g + NBUF) * CH, CH)], ids[b],
                    sems[b])

        return carry

    lax.fori_loop(0, NCHUNK // NBUF, body, 0)
    for b in range(NCHUNK % NBUF):
        g = (NCHUNK // NBUF) * NBUF + b
        pltpu.make_async_copy(dst_hbm.at[pl.ds(base, CH)], ids[b],
                              sems[b]).wait()
        pltpu.sync_copy(ones_v, acc.at[ids[b]], add=True)
    plsc.subcore_barrier()
    pltpu.sync_copy(acc.at[pl.ds(s * ROWS_PER_TILE, ROWS_PER_TILE)], stage)
    pltpu.sync_copy(stage,
                    out_hbm.at[pl.ds(c * N + s * ROWS_PER_TILE,
                                     ROWS_PER_TILE)])

    @pl.when(s == 0)
    def _():
        pltpu.sync_copy(acc.at[pl.ds(TAIL_START, TAIL_ROWS)],
                        stage.at[pl.ds(0, TAIL_ROWS)])
        pltpu.sync_copy(stage.at[pl.ds(0, TAIL_ROWS)],
                        out_hbm.at[pl.ds(c * N + TAIL_START, TAIL_ROWS)])


@functools.partial(
    pl.kernel,
    out_type=jax.ShapeDtypeStruct((NC, N, D), jnp.float32),
    mesh=_MESH,
    scratch_types=[
        pltpu.VMEM((EPW,), jnp.int32),
        [pltpu.VMEM((CH,), jnp.int32)] * ABUF,
        [pltpu.VMEM((CH, D), jnp.float32)] * ABUF,
        pltpu.VMEM_SHARED((N, D), jnp.float32),
        [pltpu.SemaphoreType.DMA] * ABUF,
        [pltpu.SemaphoreType.DMA] * ABUF,
    ],
)
def _agg_kernel(src_hbm, dst_hbm, y_hbm, zeros_hbm, out_hbm,
                src_v, ids, bufs, acc, semg, semi):
    c = lax.axis_index("c")
    s = lax.axis_index("s")
    wid = c * NS + s
    base = wid * EPW
    pltpu.sync_copy(src_hbm.at[pl.ds(base, EPW)], src_v)
    pltpu.sync_copy(zeros_hbm.at[pl.ds(0, CH)], bufs[0])
    for k in range(ROWS_PER_TILE // CH):
        pltpu.sync_copy(bufs[0],
                        acc.at[pl.ds(s * ROWS_PER_TILE + k * CH, CH)])
    pltpu.sync_copy(
        bufs[0].at[pl.ds(0, ROWS_PER_TILE % CH)],
        acc.at[pl.ds(s * ROWS_PER_TILE + (ROWS_PER_TILE // CH) * CH,
                     ROWS_PER_TILE % CH)])

    @pl.when(s == 0)
    def _():
        pltpu.sync_copy(bufs[0].at[pl.ds(0, TAIL_ROWS)],
                        acc.at[pl.ds(TAIL_START, TAIL_ROWS)])

    plsc.subcore_barrier()

    for b in range(ABUF):
        pltpu.async_copy(y_hbm.at[src_v.at[pl.ds(b * CH, CH)]], bufs[b],
                         semg[b])
        pltpu.async_copy(dst_hbm.at[pl.ds(base + b * CH, CH)], ids[b],
                         semi[b])

    def body(k, carry):
        for b in range(ABUF):
            g = ABUF * k + b
            pltpu.make_async_copy(y_hbm.at[src_v.at[pl.ds(0, CH)]], bufs[b],
                                  semg[b]).wait()
            pltpu.make_async_copy(dst_hbm.at[pl.ds(base, CH)], ids[b],
                                  semi[b]).wait()
            pltpu.sync_copy(bufs[b], acc.at[ids[b]], add=True)

            @pl.when(g + ABUF < NCHUNK)
            def _():
                pltpu.async_copy(
                    y_hbm.at[src_v.at[pl.ds((g + ABUF) * CH, CH)]], bufs[b],
                    semg[b])
                pltpu.async_copy(
                    dst_hbm.at[pl.ds(base + (g + ABUF) * CH, CH)], ids[b],
                    semi[b])

        return carry

    lax.fori_loop(0, NCHUNK // ABUF, body, 0)
    for b in range(NCHUNK % ABUF):
        pltpu.make_async_copy(y_hbm.at[src_v.at[pl.ds(0, CH)]], bufs[b],
                              semg[b]).wait()
        pltpu.make_async_copy(dst_hbm.at[pl.ds(base, CH)], ids[b],
                              semi[b]).wait()
        pltpu.sync_copy(bufs[b], acc.at[ids[b]], add=True)
    plsc.subcore_barrier()
    for k in range(ROWS_PER_TILE // CH):
        b = k % ABUF
        r0 = s * ROWS_PER_TILE + k * CH
        pltpu.sync_copy(acc.at[pl.ds(r0, CH)], bufs[b])
        pltpu.sync_copy(bufs[b], out_hbm.at[c, pl.ds(r0, CH)])
    rrem = ROWS_PER_TILE % CH
    rbase = s * ROWS_PER_TILE + (ROWS_PER_TILE // CH) * CH
    pltpu.sync_copy(acc.at[pl.ds(rbase, rrem)], bufs[0].at[pl.ds(0, rrem)])
    pltpu.sync_copy(bufs[0].at[pl.ds(0, rrem)],
                    out_hbm.at[c, pl.ds(rbase, rrem)])

    @pl.when(s == 0)
    def _():
        pltpu.sync_copy(acc.at[pl.ds(TAIL_START, TAIL_ROWS)],
                        bufs[1].at[pl.ds(0, TAIL_ROWS)])
        pltpu.sync_copy(bufs[1].at[pl.ds(0, TAIL_ROWS)],
                        out_hbm.at[c, pl.ds(TAIL_START, TAIL_ROWS)])


_GRID = 5
_R = N // _GRID


def _prep_body(hist_ref, x_ref, mask_ref, y1_ref, u_ref):
    deg = hist_ref[:, 0] + hist_ref[:, 1] + 1.0
    u = lax.rsqrt(deg)
    ub = jnp.broadcast_to(u[:, None], (_R, D))
    u_ref[...] = ub
    y1_ref[...] = x_ref[...] * mask_ref[...] * ub


def _tc_prep(hist_t, x, mask):
    return pl.pallas_call(
        _prep_body,
        grid=(_GRID,),
        in_specs=[
            pl.BlockSpec((_R, NC), lambda i: (i, 0)),
            pl.BlockSpec((_R, D), lambda i: (i, 0)),
            pl.BlockSpec((_R, D), lambda i: (i, 0)),
        ],
        out_specs=[
            pl.BlockSpec((_R, D), lambda i: (i, 0)),
            pl.BlockSpec((_R, D), lambda i: (i, 0)),
        ],
        out_shape=[
            jax.ShapeDtypeStruct((N, D), jnp.float32),
            jax.ShapeDtypeStruct((N, D), jnp.float32),
        ],
    )(hist_t, x, mask)


def _dense_body(s_ref, y1_ref, u_ref, w1_ref, b1_ref, w2_ref, y2_ref):
    t = (s_ref[0] + s_ref[1] + y1_ref[...]) * u_ref[...]
    z = jnp.dot(t, w1_ref[...], preferred_element_type=jnp.float32,
                precision=lax.Precision.DEFAULT)
    z = jnp.maximum(z + b1_ref[...][None, :], 0.0)
    h2 = jnp.dot(z, w2_ref[...], preferred_element_type=jnp.float32,
                 precision=lax.Precision.DEFAULT)
    y2_ref[...] = h2 * u_ref[...]


def _tc_dense(s_sum, y1, u, w1, b1, w2):
    return pl.pallas_call(
        _dense_body,
        grid=(_GRID,),
        in_specs=[
            pl.BlockSpec((NC, _R, D), lambda i: (0, i, 0)),
            pl.BlockSpec((_R, D), lambda i: (i, 0)),
            pl.BlockSpec((_R, D), lambda i: (i, 0)),
            pl.BlockSpec((D, H), lambda i: (0, 0)),
            pl.BlockSpec((H,), lambda i: (0,)),
            pl.BlockSpec((H, D), lambda i: (0, 0)),
        ],
        out_specs=pl.BlockSpec((_R, D), lambda i: (i, 0)),
        out_shape=jax.ShapeDtypeStruct((N, D), jnp.float32),
    )(s_sum, y1, u, w1, b1, w2)


def _final_body(s_ref, y2_ref, u_ref, b2_ref, out_ref):
    t = (s_ref[0] + s_ref[1] + y2_ref[...]) * u_ref[...]
    out_ref[...] = t + b2_ref[...][None, :]


def _tc_final(s2_sum, y2, u, b2):
    return pl.pallas_call(
        _final_body,
        grid=(_GRID,),
        in_specs=[
            pl.BlockSpec((NC, _R, D), lambda i: (0, i, 0)),
            pl.BlockSpec((_R, D), lambda i: (i, 0)),
            pl.BlockSpec((_R, D), lambda i: (i, 0)),
            pl.BlockSpec((D,), lambda i: (0,)),
        ],
        out_specs=pl.BlockSpec((_R, D), lambda i: (i, 0)),
        out_shape=jax.ShapeDtypeStruct((N, D), jnp.float32),
    )(s2_sum, y2, u, b2)


def kernel(x, edge_index, input_mask, W1, b1, W2, b2):
    src = edge_index[0]
    dst = edge_index[1]
    zeros_n = jnp.zeros((N,), jnp.float32)
    zeros_nd = jnp.zeros((N, D), jnp.float32)

    hist = _deg_kernel(dst, zeros_n)
    y1, u = _tc_prep(hist.reshape(NC, N).T, x, input_mask)
    s1 = _agg_kernel(src, dst, y1, zeros_nd)
    y2 = _tc_dense(s1, y1, u, W1, b1, W2)
    s2 = _agg_kernel(src, dst, y2, zeros_nd)
    return _tc_final(s2, y2, u, b2)


# speedup vs baseline: 49.8761x; 1.0222x over previous
import functools

import jax
import jax.numpy as jnp
from jax import lax
from jax.experimental import pallas as pl
from jax.experimental.pallas import tpu as pltpu
from jax.experimental.pallas import tpu_sc as plsc

N = 10000
D = 128
H = 640
E = 320000

NC = 2
NS = 16
NW = NC * NS

CH = 80
EPW = E // NW
NCHUNK = EPW // CH

ROWS_PER_TILE = 624
TAIL_START = ROWS_PER_TILE * NS
TAIL_ROWS = N - TAIL_START
NBUF = 8
ABUF = 3

_MESH = plsc.VectorSubcoreMesh(
    core_axis_name="c", subcore_axis_name="s", num_cores=NC, num_subcores=NS
)


@functools.partial(
    pl.kernel,
    out_type=jax.ShapeDtypeStruct((NC * N,), jnp.float32),
    mesh=_MESH,
    scratch_types=[
        [pltpu.VMEM((CH,), jnp.int32)] * NBUF,
        pltpu.VMEM((CH,), jnp.float32),
        pltpu.VMEM((ROWS_PER_TILE,), jnp.float32),
        pltpu.VMEM_SHARED((N,), jnp.float32),
        [pltpu.SemaphoreType.DMA] * NBUF,
    ],
)
def _deg_kernel(dst_hbm, zeros_hbm, out_hbm, ids, ones_v, stage, acc, sems):
    c = lax.axis_index("c")
    s = lax.axis_index("s")
    wid = c * NS + s
    base = wid * EPW
    for i in range(CH // 16):
        ones_v[pl.ds(i * 16, 16)] = jnp.ones((16,), jnp.float32)
    pltpu.sync_copy(zeros_hbm.at[pl.ds(0, ROWS_PER_TILE)], stage)
    pltpu.sync_copy(stage, acc.at[pl.ds(s * ROWS_PER_TILE, ROWS_PER_TILE)])

    @pl.when(s == 0)
    def _():
        pltpu.sync_copy(stage.at[pl.ds(0, TAIL_ROWS)],
                        acc.at[pl.ds(TAIL_START, TAIL_ROWS)])

    plsc.subcore_barrier()
    for b in range(NBUF):
        pltpu.async_copy(dst_hbm.at[pl.ds(base + b * CH, CH)], ids[b],
                         sems[b])

    def body(k, carry):
        for b in range(NBUF):
            g = NBUF * k + b
            pltpu.make_async_copy(dst_hbm.at[pl.ds(base, CH)], ids[b],
                                  sems[b]).wait()
            pltpu.sync_copy(ones_v, acc.at[ids[b]], add=True)

            @pl.when(g + NBUF < NCHUNK)
            def _():
                pltpu.async_copy(
                    dst_hbm.at[pl.ds(base + (g + NBUF) * CH, CH)], ids[b],
                    sems[b])

        return carry

    lax.fori_loop(0, NCHUNK // NBUF, body, 0)
    for b in range(NCHUNK % NBUF):
        g = (NCHUNK // NBUF) * NBUF + b
        pltpu.make_async_copy(dst_hbm.at[pl.ds(base, CH)], ids[b],
                              sems[b]).wait()
        pltpu.sync_copy(ones_v, acc.at[ids[b]], add=True)
    plsc.subcore_barrier()
    pltpu.sync_copy(acc.at[pl.ds(s * ROWS_PER_TILE, ROWS_PER_TILE)], stage)
    pltpu.sync_copy(stage,
                    out_hbm.at[pl.ds(c * N + s * ROWS_PER_TILE,
                                     ROWS_PER_TILE)])

    @pl.when(s == 0)
    def _():
        pltpu.sync_copy(acc.at[pl.ds(TAIL_START, TAIL_ROWS)],
                        stage.at[pl.ds(0, TAIL_ROWS)])
        pltpu.sync_copy(stage.at[pl.ds(0, TAIL_ROWS)],
                        out_hbm.at[pl.ds(c * N + TAIL_START, TAIL_ROWS)])


@functools.partial(
    pl.kernel,
    out_type=jax.ShapeDtypeStruct((NC, N, D), jnp.float32),
    mesh=_MESH,
    scratch_types=[
        pltpu.VMEM((EPW,), jnp.int32),
        [pltpu.VMEM((CH,), jnp.int32)] * ABUF,
        [pltpu.VMEM((CH, D), jnp.float32)] * ABUF,
        pltpu.VMEM_SHARED((N, D), jnp.float32),
        [pltpu.SemaphoreType.DMA] * ABUF,
        [pltpu.SemaphoreType.DMA] * ABUF,
        pltpu.SemaphoreType.DMA,
    ],
)
def _agg_kernel(src_hbm, dst_hbm, y_hbm, zeros_hbm, out_hbm,
                src_v, ids, bufs, acc, semg, semi, sems):
    c = lax.axis_index("c")
    s = lax.axis_index("s")
    wid = c * NS + s
    base = wid * EPW
    pltpu.async_copy(src_hbm.at[pl.ds(base, EPW)], src_v, sems)
    pltpu.sync_copy(zeros_hbm.at[pl.ds(0, CH)], bufs[0])
    for k in range(ROWS_PER_TILE // CH):
        pltpu.sync_copy(bufs[0],
                        acc.at[pl.ds(s * ROWS_PER_TILE + k * CH, CH)])
    pltpu.sync_copy(
        bufs[0].at[pl.ds(0, ROWS_PER_TILE % CH)],
        acc.at[pl.ds(s * ROWS_PER_TILE + (ROWS_PER_TILE // CH) * CH,
                     ROWS_PER_TILE % CH)])

    @pl.when(s == 0)
    def _():
        pltpu.sync_copy(bufs[0].at[pl.ds(0, TAIL_ROWS)],
                        acc.at[pl.ds(TAIL_START, TAIL_ROWS)])

    pltpu.make_async_copy(src_hbm.at[pl.ds(base, EPW)], src_v, sems).wait()
    plsc.subcore_barrier()

    for b in range(ABUF):
        pltpu.async_copy(y_hbm.at[src_v.at[pl.ds(b * CH, CH)]], bufs[b],
                         semg[b])
        pltpu.async_copy(dst_hbm.at[pl.ds(base + b * CH, CH)], ids[b],
                         semi[b])

    def body(k, carry):
        for b in range(ABUF):
            g = ABUF * k + b
            pltpu.make_async_copy(y_hbm.at[src_v.at[pl.ds(0, CH)]], bufs[b],
                                  semg[b]).wait()
            pltpu.make_async_copy(dst_hbm.at[pl.ds(base, CH)], ids[b],
                                  semi[b]).wait()
            pltpu.sync_copy(bufs[b], acc.at[ids[b]], add=True)

            @pl.when(g + ABUF < NCHUNK)
            def _():
                pltpu.async_copy(
                    y_hbm.at[src_v.at[pl.ds((g + ABUF) * CH, CH)]], bufs[b],
                    semg[b])
                pltpu.async_copy(
                    dst_hbm.at[pl.ds(base + (g + ABUF) * CH, CH)], ids[b],
                    semi[b])

        return carry

    lax.fori_loop(0, NCHUNK // ABUF, body, 0)
    for b in range(NCHUNK % ABUF):
        pltpu.make_async_copy(y_hbm.at[src_v.at[pl.ds(0, CH)]], bufs[b],
                              semg[b]).wait()
        pltpu.make_async_copy(dst_hbm.at[pl.ds(base, CH)], ids[b],
                              semi[b]).wait()
        pltpu.sync_copy(bufs[b], acc.at[ids[b]], add=True)
    plsc.subcore_barrier()
    for k in range(ROWS_PER_TILE // CH):
        b = k % ABUF
        r0 = s * ROWS_PER_TILE + k * CH
        pltpu.sync_copy(acc.at[pl.ds(r0, CH)], bufs[b])
        pltpu.sync_copy(bufs[b], out_hbm.at[c, pl.ds(r0, CH)])
    rrem = ROWS_PER_TILE % CH
    rbase = s * ROWS_PER_TILE + (ROWS_PER_TILE // CH) * CH
    pltpu.sync_copy(acc.at[pl.ds(rbase, rrem)], bufs[0].at[pl.ds(0, rrem)])
    pltpu.sync_copy(bufs[0].at[pl.ds(0, rrem)],
                    out_hbm.at[c, pl.ds(rbase, rrem)])

    @pl.when(s == 0)
    def _():
        pltpu.sync_copy(acc.at[pl.ds(TAIL_START, TAIL_ROWS)],
                        bufs[1].at[pl.ds(0, TAIL_ROWS)])
        pltpu.sync_copy(bufs[1].at[pl.ds(0, TAIL_ROWS)],
                        out_hbm.at[c, pl.ds(TAIL_START, TAIL_ROWS)])


_GRID = 5
_R = N // _GRID


def _prep_body(hist_ref, x_ref, mask_ref, y1_ref, u_ref):
    deg = hist_ref[:, 0] + hist_ref[:, 1] + 1.0
    u = lax.rsqrt(deg)
    ub = jnp.broadcast_to(u[:, None], (_R, D))
    u_ref[...] = ub
    y1_ref[...] = x_ref[...] * mask_ref[...] * ub


def _tc_prep(hist_t, x, mask):
    return pl.pallas_call(
        _prep_body,
        grid=(_GRID,),
        in_specs=[
            pl.BlockSpec((_R, NC), lambda i: (i, 0)),
            pl.BlockSpec((_R, D), lambda i: (i, 0)),
            pl.BlockSpec((_R, D), lambda i: (i, 0)),
        ],
        out_specs=[
            pl.BlockSpec((_R, D), lambda i: (i, 0)),
            pl.BlockSpec((_R, D), lambda i: (i, 0)),
        ],
        out_shape=[
            jax.ShapeDtypeStruct((N, D), jnp.float32),
            jax.ShapeDtypeStruct((N, D), jnp.float32),
        ],
    )(hist_t, x, mask)


def _dense_body(s_ref, y1_ref, u_ref, w1_ref, b1_ref, w2_ref, y2_ref):
    t = (s_ref[0] + s_ref[1] + y1_ref[...]) * u_ref[...]
    z = jnp.dot(t, w1_ref[...], preferred_element_type=jnp.float32,
                precision=lax.Precision.DEFAULT)
    z = jnp.maximum(z + b1_ref[...][None, :], 0.0)
    h2 = jnp.dot(z, w2_ref[...], preferred_element_type=jnp.float32,
                 precision=lax.Precision.DEFAULT)
    y2_ref[...] = h2 * u_ref[...]


def _tc_dense(s_sum, y1, u, w1, b1, w2):
    return pl.pallas_call(
        _dense_body,
        grid=(_GRID,),
        in_specs=[
            pl.BlockSpec((NC, _R, D), lambda i: (0, i, 0)),
            pl.BlockSpec((_R, D), lambda i: (i, 0)),
            pl.BlockSpec((_R, D), lambda i: (i, 0)),
            pl.BlockSpec((D, H), lambda i: (0, 0)),
            pl.BlockSpec((H,), lambda i: (0,)),
            pl.BlockSpec((H, D), lambda i: (0, 0)),
        ],
        out_specs=pl.BlockSpec((_R, D), lambda i: (i, 0)),
        out_shape=jax.ShapeDtypeStruct((N, D), jnp.float32),
    )(s_sum, y1, u, w1, b1, w2)


def _final_body(s_ref, y2_ref, u_ref, b2_ref, out_ref):
    t = (s_ref[0] + s_ref[1] + y2_ref[...]) * u_ref[...]
    out_ref[...] = t + b2_ref[...][None, :]


def _tc_final(s2_sum, y2, u, b2):
    return pl.pallas_call(
        _final_body,
        grid=(_GRID,),
        in_specs=[
            pl.BlockSpec((NC, _R, D), lambda i: (0, i, 0)),
            pl.BlockSpec((_R, D), lambda i: (i, 0)),
            pl.BlockSpec((_R, D), lambda i: (i, 0)),
            pl.BlockSpec((D,), lambda i: (0,)),
        ],
        out_specs=pl.BlockSpec((_R, D), lambda i: (i, 0)),
        out_shape=jax.ShapeDtypeStruct((N, D), jnp.float32),
    )(s2_sum, y2, u, b2)


def kernel(x, edge_index, input_mask, W1, b1, W2, b2):
    src = edge_index[0]
    dst = edge_index[1]
    zeros_n = jnp.zeros((N,), jnp.float32)
    zeros_nd = jnp.zeros((N, D), jnp.float32)

    hist = _deg_kernel(dst, zeros_n)
    y1, u = _tc_prep(hist.reshape(NC, N).T, x, input_mask)
    s1 = _agg_kernel(src, dst, y1, zeros_nd)
    y2 = _tc_dense(s1, y1, u, W1, b1, W2)
    s2 = _agg_kernel(src, dst, y2, zeros_nd)
    return _tc_final(s2, y2, u, b2)
